# baseline (speedup 1.0000x reference)
import functools

import jax
import jax.numpy as jnp
from jax import lax
from jax.experimental import pallas as pl
from jax.experimental.pallas import tpu as pltpu
from jax.experimental.pallas import tpu_sc as plsc

B = 16384
D = 32
N_ROWS = 1000000
NC = 2
NS = 16
NW = NC * NS
B_PER_W = B // NW
IDX_CH = B_PER_W // 128
CB = 4096
N_WIDE = (pl.cdiv(N_ROWS, 512)) * 128


def _transpose_body(ut_ref, it_ref, eye_ref, u_ref, i_ref):
  def pack(x):
    outs = []
    for g in range(CB // 512):
      c0 = g * 512
      stacked = jnp.concatenate(
          [x[:, c0 + 128 * a: c0 + 128 * (a + 1)] for a in range(4)], axis=0)
      outs.append(jnp.swapaxes(stacked, 0, 1))
    return jnp.concatenate(outs, axis=0)

  u_ref[...] = pack(ut_ref[...]).astype(jnp.bfloat16)
  i_ref[...] = pack(it_ref[...]).astype(jnp.bfloat16)


def _tc_transpose(utab_t, itab_t):
  grid = (pl.cdiv(N_ROWS, CB),)
  return pl.pallas_call(
      _transpose_body,
      grid=grid,
      in_specs=[
          pl.BlockSpec((D, CB), lambda i: (0, i)),
          pl.BlockSpec((D, CB), lambda i: (0, i)),
          pl.BlockSpec((D, D), lambda i: (0, 0)),
      ],
      out_specs=[
          pl.BlockSpec((CB // 4, 128), lambda i: (i, 0)),
          pl.BlockSpec((CB // 4, 128), lambda i: (i, 0)),
      ],
      out_shape=[
          jax.ShapeDtypeStruct((N_WIDE, 128), jnp.bfloat16),
          jax.ShapeDtypeStruct((N_WIDE, 128), jnp.bfloat16),
      ],
  )(utab_t, itab_t, jnp.eye(D, dtype=jnp.float32))


def _sc_gather(uids, iids, user_table, item_table):
  mesh = plsc.VectorSubcoreMesh(
      core_axis_name="c", subcore_axis_name="s", num_cores=NC, num_subcores=NS
  )

  @functools.partial(
      pl.kernel,
      out_type=[
          jax.ShapeDtypeStruct((B, 128), jnp.bfloat16),
          jax.ShapeDtypeStruct((B, 128), jnp.bfloat16),
      ],
      mesh=mesh,
      compiler_params=pltpu.CompilerParams(use_tc_tiling_on_sc=False),
      scratch_types=[
          pltpu.VMEM((IDX_CH, 128), jnp.int32),
          pltpu.VMEM((IDX_CH, 128), jnp.int32),
          pltpu.VMEM((B_PER_W, 128), jnp.bfloat16),
          pltpu.SemaphoreType.DMA,
      ],
  )
  def gather_kernel(uids_hbm, iids_hbm, utab_hbm, itab_hbm, out_u, out_i,
                    ids_v, wrow_v, wide_v, sem):
    wid = lax.axis_index("s") * NC + lax.axis_index("c")
    base = wid * B_PER_W

    def one_table(ids_hbm, tab_hbm, out_hbm):
      pltpu.sync_copy(ids_hbm.at[wid], ids_v)

      for t in range(IDX_CH * 8):
        ch, k = t // 8, t % 8
        v = ids_v.at[ch][pl.ds(k * 16, 16)]
        wrow_v.at[ch][pl.ds(k * 16, 16)] = ((v >> 9) << 7) | (v & 127)

      copies = []
      for j in range(IDX_CH):
        copies.append(pltpu.async_copy(
            tab_hbm.at[wrow_v.at[j]], wide_v.at[pl.ds(j * 128, 128)], sem))
      for c in copies:
        c.wait()
      pltpu.sync_copy(wide_v, out_hbm.at[pl.ds(base, B_PER_W)])

    one_table(uids_hbm, utab_hbm, out_u)
    one_table(iids_hbm, itab_hbm, out_i)

  return gather_kernel(uids, iids, user_table, item_table)


def _mlp_body(u_ref, v_ref, idu_ref, idv_ref, w1u_ref, w1v_ref, b1_ref,
              w2_ref, b2_ref, w3_ref, b3_ref, o_ref):
  bs = u_ref.shape[0]
  lanes = jax.lax.broadcasted_iota(jnp.int32, (bs, 128), 1)

  def masked(x, ids):
    lb = (((ids >> 7) & 3) << 5)[:, None]
    keep = (lanes >= lb) & (lanes < lb + D)
    return jnp.where(keep, x.astype(jnp.float32), 0.0)

  u = masked(u_ref[...], idu_ref[...])
  v = masked(v_ref[...], idv_ref[...])
  h = u @ w1u_ref[...] + v @ w1v_ref[...] + b1_ref[...]
  h = jnp.maximum(h, 0.0)
  h2 = jnp.maximum(h @ w2_ref[...] + b2_ref[...], 0.0)
  o_ref[...] = jnp.sum(h2 * w3_ref[...], axis=1) + b3_ref[...]


def _tc_mlp(wide_u, wide_i, idu, idv, w1u_s, w1v_s, b1, w2, b2, w3, b3,
            block_b=2048):
  grid = (B // block_b,)
  return pl.pallas_call(
      _mlp_body,
      grid=grid,
      in_specs=[
          pl.BlockSpec((block_b, 128), lambda i: (i, 0)),
          pl.BlockSpec((block_b, 128), lambda i: (i, 0)),
          pl.BlockSpec((block_b,), lambda i: (i,)),
          pl.BlockSpec((block_b,), lambda i: (i,)),
          pl.BlockSpec((128, 64), lambda i: (0, 0)),
          pl.BlockSpec((128, 64), lambda i: (0, 0)),
          pl.BlockSpec((64,), lambda i: (0,)),
          pl.BlockSpec((64, 16), lambda i: (0, 0)),
          pl.BlockSpec((16,), lambda i: (0,)),
          pl.BlockSpec((1, 16), lambda i: (0, 0)),
          pl.BlockSpec((1,), lambda i: (0,)),
      ],
      out_specs=pl.BlockSpec((block_b,), lambda i: (i,)),
      out_shape=jax.ShapeDtypeStruct((B,), jnp.float32),
  )(wide_u, wide_i, idu, idv, w1u_s, w1v_s, b1, w2, b2, w3, b3)


def kernel(user_ids, item_ids, user_table, item_table, W1, b1, W2, b2, W3, b3):
  idu = user_ids.astype(jnp.int32)
  idv = item_ids.astype(jnp.int32)
  uids = idu.reshape(NW, IDX_CH, 128)
  iids = idv.reshape(NW, IDX_CH, 128)
  tabu_w, tabi_w = _tc_transpose(user_table.T, item_table.T)
  wide_u, wide_i = _sc_gather(uids, iids, tabu_w, tabi_w)
  w1u_s = jnp.tile(W1[:, :D].T, (4, 1))
  w1v_s = jnp.tile(W1[:, D:].T, (4, 1))
  return _tc_mlp(wide_u, wide_i, idu, idv, w1u_s, w1v_s, b1, W2.T, b2, W3, b3)


# speedup vs baseline: 2.2854x; 2.2854x over previous
import functools

import jax
import jax.numpy as jnp
from jax import lax
from jax.experimental import pallas as pl
from jax.experimental.pallas import tpu as pltpu
from jax.experimental.pallas import tpu_sc as plsc

B = 16384
D = 32
N_ROWS = 1000000
NC = 2
NS = 16
NW = NC * NS
B_PER_W = B // NW
IDX_CH = B_PER_W // 128
CB = 4096
N_WIDE = (pl.cdiv(N_ROWS, 512)) * 128


def _transpose_body(ut_ref, it_ref, u_ref, i_ref):
  def pack(x):
    outs = []
    for g in range(CB // 512):
      c0 = g * 512
      stacked = jnp.concatenate(
          [x[:, c0 + 128 * a: c0 + 128 * (a + 1)] for a in range(4)], axis=0)
      outs.append(jnp.swapaxes(stacked, 0, 1))
    return jnp.concatenate(outs, axis=0)

  u_ref[...] = pack(ut_ref[...])
  i_ref[...] = pack(it_ref[...])


def _tc_transpose(utab_t, itab_t):
  grid = (pl.cdiv(N_ROWS, CB),)
  return pl.pallas_call(
      _transpose_body,
      grid=grid,
      in_specs=[
          pl.BlockSpec((D, CB), lambda i: (0, i)),
          pl.BlockSpec((D, CB), lambda i: (0, i)),
      ],
      out_specs=[
          pl.BlockSpec((CB // 4, 128), lambda i: (i, 0)),
          pl.BlockSpec((CB // 4, 128), lambda i: (i, 0)),
      ],
      out_shape=[
          jax.ShapeDtypeStruct((N_WIDE, 128), jnp.float32),
          jax.ShapeDtypeStruct((N_WIDE, 128), jnp.float32),
      ],
  )(utab_t, itab_t)


def _sc_gather(uids, iids, user_table, item_table):
  mesh = plsc.VectorSubcoreMesh(
      core_axis_name="c", subcore_axis_name="s", num_cores=NC, num_subcores=NS
  )

  @functools.partial(
      pl.kernel,
      out_type=[
          jax.ShapeDtypeStruct((B, 128), jnp.float32),
          jax.ShapeDtypeStruct((B, 128), jnp.float32),
      ],
      mesh=mesh,
      compiler_params=pltpu.CompilerParams(use_tc_tiling_on_sc=False),
      scratch_types=[
          pltpu.VMEM((IDX_CH, 128), jnp.int32),
          pltpu.VMEM((IDX_CH, 128), jnp.int32),
          pltpu.VMEM((B_PER_W, 128), jnp.float32),
          pltpu.SemaphoreType.DMA,
      ],
  )
  def gather_kernel(uids_hbm, iids_hbm, utab_hbm, itab_hbm, out_u, out_i,
                    ids_v, wrow_v, wide_v, sem):
    wid = lax.axis_index("s") * NC + lax.axis_index("c")
    base = wid * B_PER_W

    def one_table(ids_hbm, tab_hbm, out_hbm):
      pltpu.sync_copy(ids_hbm.at[wid], ids_v)

      for t in range(IDX_CH * 8):
        ch, k = t // 8, t % 8
        v = ids_v.at[ch][pl.ds(k * 16, 16)]
        wrow_v.at[ch][pl.ds(k * 16, 16)] = ((v >> 9) << 7) | (v & 127)

      copies = []
      for j in range(IDX_CH):
        copies.append(pltpu.async_copy(
            tab_hbm.at[wrow_v.at[j]], wide_v.at[pl.ds(j * 128, 128)], sem))
      for c in copies:
        c.wait()
      pltpu.sync_copy(wide_v, out_hbm.at[pl.ds(base, B_PER_W)])

    one_table(uids_hbm, utab_hbm, out_u)
    one_table(iids_hbm, itab_hbm, out_i)

  return gather_kernel(uids, iids, user_table, item_table)


def _mlp_body(u_ref, v_ref, idu_ref, idv_ref, w1u_ref, w1v_ref, b1_ref,
              w2_ref, b2_ref, w3_ref, b3_ref, o_ref):
  bs = u_ref.shape[0]
  lanes = jax.lax.broadcasted_iota(jnp.int32, (bs, 128), 1)

  def masked(x, ids):
    lb = (((ids >> 7) & 3) << 5)[:, None]
    keep = (lanes >= lb) & (lanes < lb + D)
    return jnp.where(keep, x, 0.0)

  u = masked(u_ref[...], idu_ref[...])
  v = masked(v_ref[...], idv_ref[...])
  h = u @ w1u_ref[...] + v @ w1v_ref[...] + b1_ref[...]
  h = jnp.maximum(h, 0.0)
  h2 = jnp.maximum(h @ w2_ref[...] + b2_ref[...], 0.0)
  o_ref[...] = jnp.sum(h2 * w3_ref[...], axis=1) + b3_ref[...]


def _tc_mlp(wide_u, wide_i, idu, idv, w1u_s, w1v_s, b1, w2, b2, w3, b3,
            block_b=2048):
  grid = (B // block_b,)
  return pl.pallas_call(
      _mlp_body,
      grid=grid,
      in_specs=[
          pl.BlockSpec((block_b, 128), lambda i: (i, 0)),
          pl.BlockSpec((block_b, 128), lambda i: (i, 0)),
          pl.BlockSpec((block_b,), lambda i: (i,)),
          pl.BlockSpec((block_b,), lambda i: (i,)),
          pl.BlockSpec((128, 64), lambda i: (0, 0)),
          pl.BlockSpec((128, 64), lambda i: (0, 0)),
          pl.BlockSpec((64,), lambda i: (0,)),
          pl.BlockSpec((64, 16), lambda i: (0, 0)),
          pl.BlockSpec((16,), lambda i: (0,)),
          pl.BlockSpec((1, 16), lambda i: (0, 0)),
          pl.BlockSpec((1,), lambda i: (0,)),
      ],
      out_specs=pl.BlockSpec((block_b,), lambda i: (i,)),
      out_shape=jax.ShapeDtypeStruct((B,), jnp.float32),
  )(wide_u, wide_i, idu, idv, w1u_s, w1v_s, b1, w2, b2, w3, b3)


def kernel(user_ids, item_ids, user_table, item_table, W1, b1, W2, b2, W3, b3):
  idu = user_ids.astype(jnp.int32)
  idv = item_ids.astype(jnp.int32)
  uids = idu.reshape(NW, IDX_CH, 128)
  iids = idv.reshape(NW, IDX_CH, 128)
  tabu_w, tabi_w = _tc_transpose(user_table.T, item_table.T)
  wide_u, wide_i = _sc_gather(uids, iids, tabu_w, tabi_w)
  w1u_s = jnp.tile(W1[:, :D].T, (4, 1))
  w1v_s = jnp.tile(W1[:, D:].T, (4, 1))
  return _tc_mlp(wide_u, wide_i, idu, idv, w1u_s, w1v_s, b1, W2.T, b2, W3, b3)
